# baseline (speedup 1.0000x reference)
import functools
import math

import jax
import jax.numpy as jnp
from jax import lax
from jax.experimental import pallas as pl
from jax.experimental.pallas import tpu as pltpu
from jax.experimental.pallas import tpu_sc as plsc

VOCAB = 1000000
EMB = 64
B = 1024
L = 200
N = B * L

NC = 2
NS = 16
NW = NC * NS
NIDX = N // NW
CHUNK = 64
NCHUNK = NIDX // CHUNK
NBUF = 4
LOOKAHEAD = 2
NGROUP = NCHUNK // NBUF
SCALE = math.sqrt(EMB)


def _body(tok_hbm, table_hbm, out_hbm, idx_v, bufs, gsems, ssems):
    wid = lax.axis_index("s") * NC + lax.axis_index("c")

    pltpu.sync_copy(tok_hbm.at[wid], idx_v)

    def gather_start(j, b):
        pltpu.async_copy(table_hbm.at[idx_v.at[j]], bufs[b], gsems[b])

    def gather_wait(j, b):
        pltpu.make_async_copy(table_hbm.at[idx_v.at[j]], bufs[b],
                              gsems[b]).wait()

    def store_start(j, b):
        pltpu.async_copy(bufs[b], out_hbm.at[wid, j], ssems[b])

    def store_wait(j, b):
        pltpu.make_async_copy(bufs[b], out_hbm.at[wid, j], ssems[b]).wait()

    def scale(b):
        buf = bufs[b]

        @pl.loop(0, CHUNK, unroll=8)
        def _(r):
            for c4 in range(EMB // 16):
                sl = (r, pl.ds(c4 * 16, 16))
                buf[sl] = buf[sl] * SCALE

    for j in range(LOOKAHEAD):
        gather_start(j, j)

    @pl.loop(0, NGROUP)
    def _(g):
        for b in range(NBUF):
            j = g * NBUF + b
            f = j + LOOKAHEAD
            fb = (b + LOOKAHEAD) % NBUF

            @pl.when(f < NCHUNK)
            def _():
                @pl.when(f >= NBUF)
                def _():
                    store_wait(f - NBUF, fb)

                gather_start(f, fb)

            gather_wait(j, b)
            scale(b)
            store_start(j, b)

    for j in range(NCHUNK - LOOKAHEAD, NCHUNK):
        store_wait(j, j % NBUF)


@functools.partial(jax.jit, static_argnames=())
def kernel(tokens, table):
    tok3d = tokens.reshape(NW, NCHUNK, CHUNK).astype(jnp.int32)

    sc_gather = pl.kernel(
        _body,
        out_type=jax.ShapeDtypeStruct((NW, NCHUNK, CHUNK, EMB), jnp.float32),
        mesh=plsc.VectorSubcoreMesh(core_axis_name="c", subcore_axis_name="s"),
        scratch_types=dict(
            idx_v=pltpu.VMEM((NCHUNK, CHUNK), jnp.int32),
            bufs=[pltpu.VMEM((CHUNK, EMB), jnp.float32) for _ in range(NBUF)],
            gsems=[pltpu.SemaphoreType.DMA for _ in range(NBUF)],
            ssems=[pltpu.SemaphoreType.DMA for _ in range(NBUF)],
        ),
        compiler_params=pltpu.CompilerParams(use_tc_tiling_on_sc=False),
    )
    out = sc_gather(tok3d, table)
    return out.reshape(B, L, EMB)


# speedup vs baseline: 1.0084x; 1.0084x over previous
import functools
import math

import jax
import jax.numpy as jnp
from jax import lax
from jax.experimental import pallas as pl
from jax.experimental.pallas import tpu as pltpu
from jax.experimental.pallas import tpu_sc as plsc

VOCAB = 1000000
EMB = 64
B = 1024
L = 200

NC = 2
NS = 16
NW = NC * NS
ROWS = B // NW
CHUNK = 200
HALVES = L // CHUNK
NCHUNK = ROWS * HALVES
NBUF = 4
LOOKAHEAD = 2
NGROUP = NCHUNK // NBUF
SCALE = math.sqrt(EMB)


def _body(tok_hbm, table_hbm, out_hbm, idx_v, bufs, gsems, ssems):
    wid = lax.axis_index("s") * NC + lax.axis_index("c")
    row0 = wid * ROWS

    pltpu.sync_copy(tok_hbm.at[pl.ds(row0, ROWS)], idx_v)

    def src(j):
        r, h = j // HALVES, j % HALVES
        return table_hbm.at[idx_v.at[r, pl.ds(h * CHUNK, CHUNK)]]

    def dst(j):
        r, h = j // HALVES, j % HALVES
        return out_hbm.at[row0 + r, pl.ds(h * CHUNK, CHUNK)]

    def gather_start(j, b):
        pltpu.async_copy(src(j), bufs[b], gsems[b])

    def gather_wait(j, b):
        pltpu.make_async_copy(src(j), bufs[b], gsems[b]).wait()

    def store_start(j, b):
        pltpu.async_copy(bufs[b], dst(j), ssems[b])

    def store_wait(j, b):
        pltpu.make_async_copy(bufs[b], dst(j), ssems[b]).wait()

    def scale(b):
        buf = bufs[b]

        @pl.loop(0, CHUNK, unroll=4)
        def _(r):
            for c4 in range(EMB // 16):
                sl = (r, pl.ds(c4 * 16, 16))
                buf[sl] = buf[sl] * SCALE

    for j in range(LOOKAHEAD):
        gather_start(j, j)

    @pl.loop(0, NGROUP)
    def _(g):
        for b in range(NBUF):
            j = g * NBUF + b
            f = j + LOOKAHEAD
            fb = (b + LOOKAHEAD) % NBUF

            @pl.when(f < NCHUNK)
            def _():
                @pl.when(f >= NBUF)
                def _():
                    store_wait(f - NBUF, fb)

                gather_start(f, fb)

            gather_wait(j, b)
            scale(b)
            store_start(j, b)

    for j in range(NCHUNK - LOOKAHEAD, NCHUNK):
        store_wait(j, j % NBUF)


@functools.partial(jax.jit, static_argnames=())
def kernel(tokens, table):
    sc_gather = pl.kernel(
        _body,
        out_type=jax.ShapeDtypeStruct((B, L, EMB), jnp.float32),
        mesh=plsc.VectorSubcoreMesh(core_axis_name="c", subcore_axis_name="s"),
        scratch_types=dict(
            idx_v=pltpu.VMEM((ROWS, L), jnp.int32),
            bufs=[pltpu.VMEM((CHUNK, EMB), jnp.float32) for _ in range(NBUF)],
            gsems=[pltpu.SemaphoreType.DMA for _ in range(NBUF)],
            ssems=[pltpu.SemaphoreType.DMA for _ in range(NBUF)],
        ),
        compiler_params=pltpu.CompilerParams(use_tc_tiling_on_sc=False),
    )
    return sc_gather(tokens.astype(jnp.int32), table)
